# baseline (speedup 1.0000x reference)
import functools

import jax
import jax.numpy as jnp
from jax import lax
from jax.experimental import pallas as pl
from jax.experimental.pallas import tpu as pltpu
from jax.experimental.pallas import tpu_sc as plsc

BATCH = 4096
HIST = 200
EMBED = 64
LANES = 128
B = BATCH * HIST

_info = plsc.get_sparse_core_info()
NC, NS = _info.num_cores, _info.num_subcores
NW = NC * NS
LPW = B // NW
CH = 2 * HIST
NCH = LPW // CH
NBUF = 2
assert NCH % NBUF == 0
assert LPW * 4 + NBUF * CH * LANES * 4 <= 524284


def _body(x_hbm, table_hbm, out_hbm, idx_v, rows, *sems):
    sg, ss = sems[:NBUF], sems[NBUF:]
    wid = lax.axis_index("s") * NC + lax.axis_index("c")
    base = wid * LPW

    pltpu.sync_copy(x_hbm.at[pl.ds(base, LPW)], idx_v)

    def start_gather(j, b):
        idx = idx_v.at[pl.ds(j * CH, CH)]
        pltpu.async_copy(table_hbm.at[idx], rows.at[b], sg[b])

    def wait_gather(b):
        pltpu.make_async_copy(table_hbm.at[pl.ds(0, CH)], rows.at[b], sg[b]).wait()

    def start_store(j, b):
        pltpu.async_copy(rows.at[b], out_hbm.at[pl.ds(base + j * CH, CH)], ss[b])

    def wait_store(b):
        pltpu.make_async_copy(rows.at[b], out_hbm.at[pl.ds(0, CH)], ss[b]).wait()

    for b in range(NBUF):
        start_gather(b, b)

    @pl.loop(0, NCH - NBUF, step=NBUF)
    def _loop(i):
        for b in range(NBUF):
            wait_gather(b)
            start_store(i + b, b)
        for b in range(NBUF):
            wait_store(b)
            start_gather(i + NBUF + b, b)

    i0 = NCH - NBUF
    for b in range(NBUF):
        wait_gather(b)
        start_store(i0 + b, b)
    for b in range(NBUF):
        wait_store(b)


_mesh = plsc.VectorSubcoreMesh(core_axis_name="c", subcore_axis_name="s")

_emb = functools.partial(
    pl.kernel,
    out_type=jax.ShapeDtypeStruct((B, LANES), jnp.float32),
    mesh=_mesh,
    scratch_types=[
        pltpu.VMEM((LPW,), jnp.int32),
        pltpu.VMEM((NBUF, CH, LANES), jnp.float32),
    ] + [pltpu.SemaphoreType.DMA] * (2 * NBUF),
    compiler_params=pltpu.CompilerParams(use_tc_tiling_on_sc=True),
)(_body)


def kernel(x, table):
    tp = jnp.pad(table, ((0, 0), (0, LANES - EMBED)))
    out = _emb(x.reshape(B).astype(jnp.int32), tp)
    return out[:, :EMBED].reshape(BATCH, HIST, EMBED)


# speedup vs baseline: 1.0134x; 1.0134x over previous
import functools

import jax
import jax.numpy as jnp
from jax import lax
from jax.experimental import pallas as pl
from jax.experimental.pallas import tpu as pltpu
from jax.experimental.pallas import tpu_sc as plsc

BATCH = 4096
HIST = 200
EMBED = 64
LANES = 128
B = BATCH * HIST

_info = plsc.get_sparse_core_info()
NC, NS = _info.num_cores, _info.num_subcores
NW = NC * NS
BPW = BATCH // NW
NCH = BPW
NBUF = 4
assert NCH % NBUF == 0
assert NCH * HIST * 4 + NBUF * HIST * LANES * 4 <= 524284


def _body(x_hbm, table_hbm, out_hbm, idx_v, rows, *sems):
    sg, ss = sems[:NBUF], sems[NBUF:]
    wid = lax.axis_index("s") * NC + lax.axis_index("c")
    base = wid * BPW

    pltpu.sync_copy(x_hbm.at[pl.ds(base * HIST, NCH * HIST)], idx_v)

    def start_gather(j, b):
        idx = idx_v.at[pl.ds(j * HIST, HIST)]
        pltpu.async_copy(table_hbm.at[idx], rows.at[b], sg[b])

    def wait_gather(b):
        pltpu.make_async_copy(table_hbm.at[pl.ds(0, HIST)], rows.at[b], sg[b]).wait()

    def start_store(j, b):
        pltpu.async_copy(rows.at[b], out_hbm.at[base + j], ss[b])

    def wait_store(b):
        pltpu.make_async_copy(rows.at[b], out_hbm.at[0], ss[b]).wait()

    for b in range(NBUF):
        start_gather(b, b)

    @pl.loop(0, NCH - NBUF, step=NBUF)
    def _loop(i):
        for b in range(NBUF):
            wait_gather(b)
            start_store(i + b, b)
        for b in range(NBUF):
            wait_store(b)
            start_gather(i + NBUF + b, b)

    i0 = NCH - NBUF
    for b in range(NBUF):
        wait_gather(b)
        start_store(i0 + b, b)
    for b in range(NBUF):
        wait_store(b)


_mesh = plsc.VectorSubcoreMesh(core_axis_name="c", subcore_axis_name="s")

_emb = functools.partial(
    pl.kernel,
    out_type=jax.ShapeDtypeStruct((BATCH, HIST, LANES), jnp.float32),
    mesh=_mesh,
    scratch_types=[
        pltpu.VMEM((NCH * HIST,), jnp.int32),
        pltpu.VMEM((NBUF, HIST, LANES), jnp.float32),
    ] + [pltpu.SemaphoreType.DMA] * (2 * NBUF),
    compiler_params=pltpu.CompilerParams(use_tc_tiling_on_sc=True),
)(_body)


def kernel(x, table):
    tp = jnp.pad(table, ((0, 0), (0, LANES - EMBED)))
    return _emb(x.reshape(B).astype(jnp.int32), tp)[:, :, :EMBED]
